# baseline (speedup 1.0000x reference)
import functools

import jax
import jax.numpy as jnp
from jax import lax
from jax.experimental import pallas as pl
from jax.experimental.pallas import tpu as pltpu
from jax.experimental.pallas import tpu_sc as plsc

_VOCAB = 1000000
_EMBED = 32
_BATCH = 4096
_NC = 2
_NT = 16
_CHUNK = _BATCH // (_NC * _NT)
_NBUF = 8

_mesh = plsc.VectorSubcoreMesh(
    core_axis_name="c", subcore_axis_name="s", num_cores=_NC)


def _slog(x):
  b = lax.bitcast_convert_type(x, jnp.int32)
  e = (b >> 23) - 127
  m = lax.bitcast_convert_type((b & 0x7FFFFF) | 0x3F800000, jnp.float32)
  adj = m > 1.4142135623730951
  m = jnp.where(adj, m * 0.5, m)
  ef = e.astype(jnp.float32) + jnp.where(adj, 1.0, 0.0)
  u = m - 1.0
  lm = jnp.float32(-1.0 / 8.0)
  for k in (7, 6, 5, 4, 3, 2, 1):
    c = (1.0 if k % 2 else -1.0) / k
    lm = lm * u + jnp.float32(c)
  lm = lm * u
  return ef * 0.6931471805599453 + lm


def _combine_body(p_ref, out_ref):
  t = [p_ref[0, 0, k] + p_ref[1, 0, k] for k in range(5)]
  total = (0.5 * _BATCH) * t[0] + t[1] * t[3] + 0.5 * t[2] * t[4]
  out_ref[...] = jnp.full((8, 128), total, jnp.float32)


def _combine_tc(parts):
  return pl.pallas_call(
      _combine_body,
      out_shape=jax.ShapeDtypeStruct((8, 128), jnp.float32),
  )(parts)


@functools.partial(
    pl.kernel,
    out_type=jax.ShapeDtypeStruct((_NC, 8, 128), jnp.float32),
    mesh=_mesh,
    compiler_params=pltpu.CompilerParams(use_tc_tiling_on_sc=True),
    scratch_types=[
        pltpu.VMEM((_BATCH + 256,), jnp.int32),
        pltpu.VMEM((_BATCH + 256,), jnp.int32),
        pltpu.VMEM((_BATCH + 256,), jnp.float32),
        pltpu.VMEM((_BATCH + 256,), jnp.float32),
        pltpu.VMEM((_NBUF, _EMBED + 1, 128), jnp.float32),
        pltpu.VMEM((_NBUF, _EMBED + 1, 128), jnp.float32),
        pltpu.VMEM((_NBUF, 2, 128), jnp.float32),
        pltpu.VMEM((_NBUF, 2, 128), jnp.float32),
        pltpu.VMEM((8, 128), jnp.float32),
        pltpu.VMEM((_NT, 8, 128), jnp.float32),
        pltpu.VMEM_SHARED((_NT, 8, 128), jnp.float32),
        pltpu.SemaphoreType.DMA,
    ],
)
def _glove_sc(i_hbm, j_hbm, co_hbm, w_hbm, wvT, wwT, bvT, bwT, out_hbm,
              idxi, idxj, lco, wgt, rvi, rvj, rbv, rbw, stage, redbuf,
              shared, sem):
  cid = lax.axis_index("c")
  sid = lax.axis_index("s")
  base = (sid * _NC + cid) * _CHUNK

  pltpu.sync_copy(i_hbm, idxi.at[pl.ds(0, _BATCH)])
  pltpu.sync_copy(j_hbm, idxj.at[pl.ds(0, _BATCH)])
  pltpu.sync_copy(co_hbm, lco.at[pl.ds(0, _BATCH)])
  pltpu.sync_copy(w_hbm, wgt.at[pl.ds(0, _BATCH)])

  lanes = lax.iota(jnp.int32, 16)

  def scal(ref, n):
    return ref[pl.ds(n, 16)][0]

  def issue(n, b):
    vi = scal(idxi, base + n)
    vj = scal(idxj, base + n)
    ti = pl.multiple_of((vi >> 7) * 128, 128)
    tj = pl.multiple_of((vj >> 7) * 128, 128)
    pltpu.async_copy(wvT.at[:, pl.ds(ti, 128)], rvi.at[b, pl.ds(0, _EMBED)], sem)
    pltpu.async_copy(wwT.at[:, pl.ds(tj, 128)], rvj.at[b, pl.ds(0, _EMBED)], sem)
    pltpu.async_copy(bvT.at[:, pl.ds(ti, 128)], rbv.at[b, pl.ds(0, 1)], sem)
    pltpu.async_copy(bwT.at[:, pl.ds(tj, 128)], rbw.at[b, pl.ds(0, 1)], sem)

  def drain(b):
    pltpu.make_async_copy(
        wvT.at[:, pl.ds(0, 128)], rvi.at[b, pl.ds(0, _EMBED)], sem).wait()
    pltpu.make_async_copy(
        wwT.at[:, pl.ds(0, 128)], rvj.at[b, pl.ds(0, _EMBED)], sem).wait()
    pltpu.make_async_copy(
        bvT.at[:, pl.ds(0, 128)], rbv.at[b, pl.ds(0, 1)], sem).wait()
    pltpu.make_async_copy(
        bwT.at[:, pl.ds(0, 128)], rbw.at[b, pl.ds(0, 1)], sem).wait()

  for b in range(_NBUF):
    issue(b, b)

  def loop_body(g, carry):
    s1, s2, s3, c1, c2 = carry
    n0 = g * _NBUF
    for b in range(_NBUF):
      n = n0 + b
      drain(b)
      vi = scal(idxi, base + n)
      vj = scal(idxj, base + n)
      li = vi & 127
      lj = vj & 127
      q = jnp.zeros((16,), jnp.float32)
      for r in range(_EMBED):
        q = q + rvi[b, r, pl.ds(li, 16)] * rvj[b, r, pl.ds(lj, 16)]
      s = q[0]
      bvs = rbv[b, 0, pl.ds(li, 16)][0]
      bws = rbw[b, 0, pl.ds(lj, 16)][0]
      w = scal(wgt, base + n)
      sl = s - _slog(scal(lco, base + n))
      t = w * sl
      s1 = s1 + t * sl
      s2 = s2 + t
      s3 = s3 + w
      c = bvs + bws
      c1 = c1 + c
      c2 = c2 + c * c
      n2 = n + _NBUF

      @pl.when(n2 < _CHUNK)
      def _():
        issue(n2, b)

    return (s1, s2, s3, c1, c2)

  z = jnp.float32(0.0)
  s1, s2, s3, c1, c2 = lax.fori_loop(
      0, _CHUNK // _NBUF, loop_body, (z, z, z, z, z))

  pvec = jnp.zeros((16,), jnp.float32)
  for k, val in enumerate((s1, s2, s3, c1, c2)):
    pvec = jnp.where(lanes == k, jnp.full((16,), val, jnp.float32), pvec)
  stage[0, pl.ds(0, 16)] = pvec
  pltpu.sync_copy(stage, shared.at[sid])
  plsc.subcore_barrier()

  @pl.when(sid == 0)
  def _():
    pltpu.sync_copy(shared, redbuf)
    acc = jnp.zeros((16,), jnp.float32)
    for t in range(_NT):
      acc = acc + redbuf[t, 0, pl.ds(0, 16)]
    stage[0, pl.ds(0, 16)] = acc
    pltpu.sync_copy(stage, out_hbm.at[cid])


def kernel(i, j, co_occur, weight, Wv, Ww, bv, bw):
  parts = _glove_sc(i, j, co_occur, weight, Wv.T, Ww.T, bv.T, bw.T)
  return _combine_tc(parts)[0, 0]


# speedup vs baseline: 1.0152x; 1.0152x over previous
import functools

import jax
import jax.numpy as jnp
from jax import lax
from jax.experimental import pallas as pl
from jax.experimental.pallas import tpu as pltpu
from jax.experimental.pallas import tpu_sc as plsc

_VOCAB = 1000000
_EMBED = 32
_BATCH = 4096
_NC = 2
_NT = 16
_CHUNK = _BATCH // (_NC * _NT)
_NBUF = 8

_mesh = plsc.VectorSubcoreMesh(
    core_axis_name="c", subcore_axis_name="s", num_cores=_NC)


def _slog(x):
  b = lax.bitcast_convert_type(x, jnp.int32)
  e = (b >> 23) - 127
  m = lax.bitcast_convert_type((b & 0x7FFFFF) | 0x3F800000, jnp.float32)
  adj = m > 1.4142135623730951
  m = jnp.where(adj, m * 0.5, m)
  ef = e.astype(jnp.float32) + jnp.where(adj, 1.0, 0.0)
  u = m - 1.0
  lm = jnp.float32(-1.0 / 8.0)
  for k in (7, 6, 5, 4, 3, 2, 1):
    c = (1.0 if k % 2 else -1.0) / k
    lm = lm * u + jnp.float32(c)
  lm = lm * u
  return ef * 0.6931471805599453 + lm


def _combine_body(p_ref, out_ref):
  t = [p_ref[0, 0, k] + p_ref[1, 0, k] for k in range(5)]
  total = (0.5 * _BATCH) * t[0] + t[1] * t[3] + 0.5 * t[2] * t[4]
  out_ref[...] = jnp.full((8, 128), total, jnp.float32)


def _combine_tc(parts):
  return pl.pallas_call(
      _combine_body,
      out_shape=jax.ShapeDtypeStruct((8, 128), jnp.float32),
  )(parts)


@functools.partial(
    pl.kernel,
    out_type=jax.ShapeDtypeStruct((_NC, 8, 128), jnp.float32),
    mesh=_mesh,
    compiler_params=pltpu.CompilerParams(use_tc_tiling_on_sc=True),
    scratch_types=[
        pltpu.VMEM((_BATCH + 256,), jnp.int32),
        pltpu.VMEM((_BATCH + 256,), jnp.int32),
        pltpu.VMEM((_BATCH + 256,), jnp.float32),
        pltpu.VMEM((_BATCH + 256,), jnp.float32),
        pltpu.VMEM((_NBUF, _EMBED + 1, 128), jnp.float32),
        pltpu.VMEM((_NBUF, _EMBED + 1, 128), jnp.float32),
        pltpu.VMEM((_NBUF, 2, 128), jnp.float32),
        pltpu.VMEM((_NBUF, 2, 128), jnp.float32),
        pltpu.VMEM((8, 128), jnp.float32),
        pltpu.VMEM((_NT, 8, 128), jnp.float32),
        pltpu.VMEM_SHARED((_NT, 8, 128), jnp.float32),
        pltpu.SemaphoreType.DMA,
    ],
)
def _glove_sc(i_hbm, j_hbm, co_hbm, w_hbm, wvT, wwT, bvT, bwT, out_hbm,
              idxi, idxj, lco, wgt, rvi, rvj, rbv, rbw, stage, redbuf,
              shared, sem):
  cid = lax.axis_index("c")
  sid = lax.axis_index("s")
  base = (sid * _NC + cid) * _CHUNK

  cp_i = pltpu.async_copy(i_hbm, idxi.at[pl.ds(0, _BATCH)], sem)
  cp_j = pltpu.async_copy(j_hbm, idxj.at[pl.ds(0, _BATCH)], sem)
  cp_co = pltpu.async_copy(co_hbm, lco.at[pl.ds(0, _BATCH)], sem)
  cp_w = pltpu.async_copy(w_hbm, wgt.at[pl.ds(0, _BATCH)], sem)
  cp_i.wait()
  cp_j.wait()

  lanes = lax.iota(jnp.int32, 16)

  def scal(ref, n):
    return ref[pl.ds(n, 16)][0]

  def issue(n, b):
    vi = scal(idxi, base + n)
    vj = scal(idxj, base + n)
    ti = pl.multiple_of((vi >> 7) * 128, 128)
    tj = pl.multiple_of((vj >> 7) * 128, 128)
    pltpu.async_copy(wvT.at[:, pl.ds(ti, 128)], rvi.at[b, pl.ds(0, _EMBED)], sem)
    pltpu.async_copy(wwT.at[:, pl.ds(tj, 128)], rvj.at[b, pl.ds(0, _EMBED)], sem)
    pltpu.async_copy(bvT.at[:, pl.ds(ti, 128)], rbv.at[b, pl.ds(0, 1)], sem)
    pltpu.async_copy(bwT.at[:, pl.ds(tj, 128)], rbw.at[b, pl.ds(0, 1)], sem)

  def drain(b):
    pltpu.make_async_copy(
        wvT.at[:, pl.ds(0, 128)], rvi.at[b, pl.ds(0, _EMBED)], sem).wait()
    pltpu.make_async_copy(
        wwT.at[:, pl.ds(0, 128)], rvj.at[b, pl.ds(0, _EMBED)], sem).wait()
    pltpu.make_async_copy(
        bvT.at[:, pl.ds(0, 128)], rbv.at[b, pl.ds(0, 1)], sem).wait()
    pltpu.make_async_copy(
        bwT.at[:, pl.ds(0, 128)], rbw.at[b, pl.ds(0, 1)], sem).wait()

  for b in range(_NBUF):
    issue(b, b)
  cp_co.wait()
  cp_w.wait()

  def loop_body(g, carry):
    s1, s2, s3, c1, c2 = carry
    n0 = g * _NBUF
    for b in range(_NBUF):
      n = n0 + b
      drain(b)
      vi = scal(idxi, base + n)
      vj = scal(idxj, base + n)
      li = vi & 127
      lj = vj & 127
      q = jnp.zeros((16,), jnp.float32)
      for r in range(_EMBED):
        q = q + rvi[b, r, pl.ds(li, 16)] * rvj[b, r, pl.ds(lj, 16)]
      s = q[0]
      bvs = rbv[b, 0, pl.ds(li, 16)][0]
      bws = rbw[b, 0, pl.ds(lj, 16)][0]
      w = scal(wgt, base + n)
      sl = s - _slog(scal(lco, base + n))
      t = w * sl
      s1 = s1 + t * sl
      s2 = s2 + t
      s3 = s3 + w
      c = bvs + bws
      c1 = c1 + c
      c2 = c2 + c * c
      n2 = n + _NBUF

      @pl.when(n2 < _CHUNK)
      def _():
        issue(n2, b)

    return (s1, s2, s3, c1, c2)

  z = jnp.float32(0.0)
  s1, s2, s3, c1, c2 = lax.fori_loop(
      0, _CHUNK // _NBUF, loop_body, (z, z, z, z, z))

  pvec = jnp.zeros((16,), jnp.float32)
  for k, val in enumerate((s1, s2, s3, c1, c2)):
    pvec = jnp.where(lanes == k, jnp.full((16,), val, jnp.float32), pvec)
  stage[0, pl.ds(0, 16)] = pvec
  pltpu.sync_copy(stage, shared.at[sid])
  plsc.subcore_barrier()

  @pl.when(sid == 0)
  def _():
    pltpu.sync_copy(shared, redbuf)
    acc = jnp.zeros((16,), jnp.float32)
    for t in range(_NT):
      acc = acc + redbuf[t, 0, pl.ds(0, 16)]
    stage[0, pl.ds(0, 16)] = acc
    pltpu.sync_copy(stage, out_hbm.at[cid])


def kernel(i, j, co_occur, weight, Wv, Ww, bv, bw):
  parts = _glove_sc(i, j, co_occur, weight, Wv.T, Ww.T, bv.T, bw.T)
  return _combine_tc(parts)[0, 0]
